# Initial kernel scaffold; baseline (speedup 1.0000x reference)
import jax
import jax.numpy as jnp
from jax.experimental import pallas as pl


def kernel(target_xyz, source_xyz, source_features, target_skip_features, W1, g1, b1, W2, g2, b2):
    raise NotImplementedError("write your pallas kernel here")


# speedup vs baseline: 14.4443x; 14.4443x over previous
import functools

import jax
import jax.numpy as jnp
from jax.experimental import pallas as pl
from jax.experimental.pallas import tpu as pltpu

NB = 256


def _dotT(x, w):
    return jax.lax.dot_general(x, w, (((1,), (1,)), ((), ())),
                               preferred_element_type=jnp.float32)


def _k1_body(tx_ref, sxT_ref, F_ref, skip_ref, W1_ref, y1_ref, stats_ref):
    S = sxT_ref.shape[2]
    NBb = tx_ref.shape[1]
    t = tx_ref[0]
    s = sxT_ref[0]
    cross = jnp.dot(t, s, preferred_element_type=jnp.float32)
    t2 = jnp.sum(t * t, axis=1, keepdims=True)
    s2 = jnp.sum(s * s, axis=0, keepdims=True)
    ds = jnp.clip(t2 + s2 - 2.0 * cross, 0.0, None)
    dd = None
    for c in range(3):
        diff = t[:, c:c + 1] - s[c:c + 1, :]
        dd = diff * diff if dd is None else dd + diff * diff
    iota = jax.lax.broadcasted_iota(jnp.int32, (NBb, S), 1)
    dm = ds
    idxs, ws = [], []
    for k in range(3):
        m = jnp.min(dm, axis=1, keepdims=True)
        ik = jnp.min(jnp.where(dm <= m, iota, S), axis=1, keepdims=True)
        selk = iota == ik
        wd = jnp.sum(jnp.where(selk, dd, 0.0), axis=1, keepdims=True)
        ws.append(1.0 / (wd + 1e-8))
        idxs.append(ik)
        if k < 2:
            dm = jnp.where(selk, jnp.inf, dm)
    rs = ws[0] + ws[1] + ws[2]
    Wm = jnp.where(iota == idxs[0], ws[0] / rs,
                   jnp.where(iota == idxs[1], ws[1] / rs,
                             jnp.where(iota == idxs[2], ws[2] / rs, 0.0)))
    interp = jnp.dot(Wm, F_ref[0], preferred_element_type=jnp.float32,
                     precision=jax.lax.Precision.HIGHEST)
    y1 = _dotT(interp, W1_ref[:, :256]) + _dotT(skip_ref[0], W1_ref[:, 256:])
    y1_ref[0] = y1
    st = jnp.concatenate([jnp.sum(y1, axis=0)[None, :],
                          jnp.sum(y1 * y1, axis=0)[None, :]], axis=0)
    first = (pl.program_id(0) == 0) & (pl.program_id(1) == 0)

    @pl.when(first)
    def _():
        stats_ref[...] = st

    @pl.when(jnp.logical_not(first))
    def _():
        stats_ref[...] += st


def _k2_body(y1_ref, a1_ref, c1_ref, W2_ref, y2_ref, stats_ref):
    z = jnp.maximum(y1_ref[0] * a1_ref[...] + c1_ref[...], 0.0)
    y2 = _dotT(z, W2_ref[...])
    y2_ref[0] = y2
    st = jnp.concatenate([jnp.sum(y2, axis=0)[None, :],
                          jnp.sum(y2 * y2, axis=0)[None, :]], axis=0)
    first = (pl.program_id(0) == 0) & (pl.program_id(1) == 0)

    @pl.when(first)
    def _():
        stats_ref[...] = st

    @pl.when(jnp.logical_not(first))
    def _():
        stats_ref[...] += st


def _k3_body(y2_ref, a2_ref, c2_ref, out_ref):
    out_ref[0] = jnp.maximum(y2_ref[0] * a2_ref[...] + c2_ref[...], 0.0)


def kernel(target_xyz, source_xyz, source_features, target_skip_features,
           W1, g1, b1, W2, g2, b2):
    B, N, _ = target_xyz.shape
    S = source_xyz.shape[1]
    C2 = source_features.shape[2]
    C1 = target_skip_features.shape[2]
    nblk = N // NB
    sxT = jnp.transpose(source_xyz, (0, 2, 1))

    y1, st1 = pl.pallas_call(
        _k1_body,
        grid=(B, nblk),
        in_specs=[
            pl.BlockSpec((1, NB, 3), lambda b, n: (b, n, 0)),
            pl.BlockSpec((1, 3, S), lambda b, n: (b, 0, 0)),
            pl.BlockSpec((1, S, C2), lambda b, n: (b, 0, 0)),
            pl.BlockSpec((1, NB, C1), lambda b, n: (b, n, 0)),
            pl.BlockSpec((256, 384), lambda b, n: (0, 0)),
        ],
        out_specs=[
            pl.BlockSpec((1, NB, 256), lambda b, n: (b, n, 0)),
            pl.BlockSpec((2, 256), lambda b, n: (0, 0)),
        ],
        out_shape=[
            jax.ShapeDtypeStruct((B, N, 256), jnp.float32),
            jax.ShapeDtypeStruct((2, 256), jnp.float32),
        ],
    )(target_xyz, sxT, source_features, target_skip_features, W1)

    cnt = float(B * N)
    mean1 = st1[0] / cnt
    var1 = st1[1] / cnt - mean1 * mean1
    a1 = g1 * jax.lax.rsqrt(var1 + 1e-5)
    c1 = b1 - mean1 * a1

    y2, st2 = pl.pallas_call(
        _k2_body,
        grid=(B, nblk),
        in_specs=[
            pl.BlockSpec((1, NB, 256), lambda b, n: (b, n, 0)),
            pl.BlockSpec((1, 256), lambda b, n: (0, 0)),
            pl.BlockSpec((1, 256), lambda b, n: (0, 0)),
            pl.BlockSpec((128, 256), lambda b, n: (0, 0)),
        ],
        out_specs=[
            pl.BlockSpec((1, NB, 128), lambda b, n: (b, n, 0)),
            pl.BlockSpec((2, 128), lambda b, n: (0, 0)),
        ],
        out_shape=[
            jax.ShapeDtypeStruct((B, N, 128), jnp.float32),
            jax.ShapeDtypeStruct((2, 128), jnp.float32),
        ],
    )(y1, a1[None, :], c1[None, :], W2)

    mean2 = st2[0] / cnt
    var2 = st2[1] / cnt - mean2 * mean2
    a2 = g2 * jax.lax.rsqrt(var2 + 1e-5)
    c2 = b2 - mean2 * a2

    out = pl.pallas_call(
        _k3_body,
        grid=(B, nblk),
        in_specs=[
            pl.BlockSpec((1, NB, 128), lambda b, n: (b, n, 0)),
            pl.BlockSpec((1, 128), lambda b, n: (0, 0)),
            pl.BlockSpec((1, 128), lambda b, n: (0, 0)),
        ],
        out_specs=pl.BlockSpec((1, NB, 128), lambda b, n: (b, n, 0)),
        out_shape=jax.ShapeDtypeStruct((B, N, 128), jnp.float32),
    )(y2, a2[None, :], c2[None, :])
    return out
